# baseline (speedup 1.0000x reference)
import functools

import jax
import jax.numpy as jnp
from jax.experimental import pallas as pl
from jax.experimental.pallas import tpu as pltpu

SLOPE = 0.1
BF = jnp.bfloat16
F32 = jnp.float32


def _argnames():
    ns = []
    for s in ("spa1", "spa2", "spa3"):
        ns += [s + "_w", s + "_b"]
    for s in ("s1", "s2", "s3"):
        for al in ("align1", "align2"):
            p = s + "_" + al + "_"
            ns += [p + "conv01_w", p + "conv01_b"]
            for cc in ("c1", "c2", "c3", "cr"):
                ns += [p + "aspp_" + cc + "_w", p + "aspp_" + cc + "_b"]
            ns += [p + "conv02_w", p + "conv02_b",
                   p + "deform_w", p + "deform_b"]
        for t in ("fuse", "aux_ang1", "aux_ang2", "ang", "fuse0",
                  "embed_c1", "embed_c2"):
            ns += [s + "_" + t + "_w", s + "_" + t + "_b"]
    ns += ["in_x", "aux_spa1_0", "aux_spa1_1", "aux_spa1_2",
           "aux_spa2_0", "aux_spa2_1", "aux_spa2_2"]
    return tuple(ns)


_NAMES = _argnames()


def _f1x1(w):
    return w.reshape(w.shape[0], w.shape[1]).T.astype(BF)


def _fconv(w):
    co, ci, kh, kw = w.shape
    return jnp.transpose(w, (2, 3, 1, 0)).reshape(kh * kw * ci, co).astype(BF)


def _patches(x, kh, kw, stride=1, pad=0):
    n, h, w, c = x.shape
    xp = jnp.pad(x, ((0, 0), (pad, pad), (pad, pad), (0, 0)))
    ho = (h + 2 * pad - kh) // stride + 1
    wo = (w + 2 * pad - kw) // stride + 1
    taps = [xp[:, i:i + stride * (ho - 1) + 1:stride,
               j:j + stride * (wo - 1) + 1:stride, :].reshape(n * ho * wo, c)
            for i in range(kh) for j in range(kw)]
    return jnp.concatenate(taps, axis=1), ho, wo


def _mm_kernel(x_ref, w_ref, b_ref, o_ref, *, act):
    acc = jnp.dot(x_ref[0], w_ref[0], preferred_element_type=F32) + b_ref[0]
    if act:
        acc = jnp.where(acc >= 0, acc, SLOPE * acc)
    o_ref[0] = acc.astype(o_ref.dtype)


def _mm(x, w, b, act=False, out_dtype=F32, tm=512):
    g, m, k = x.shape
    n = w.shape[-1]
    tmm = min(m, tm)
    return pl.pallas_call(
        functools.partial(_mm_kernel, act=act),
        out_shape=jax.ShapeDtypeStruct((g, m, n), out_dtype),
        grid=(g, pl.cdiv(m, tmm)),
        in_specs=[
            pl.BlockSpec((1, tmm, k), lambda gi, i: (gi, i, 0)),
            pl.BlockSpec((1, k, n), lambda gi, i: (gi, 0, 0)),
            pl.BlockSpec((1, 1, n), lambda gi, i: (gi, 0, 0)),
        ],
        out_specs=pl.BlockSpec((1, tmm, n), lambda gi, i: (gi, i, 0)),
        compiler_params=pltpu.CompilerParams(
            dimension_semantics=("parallel", "parallel")),
    )(x, w, b)


def _rot(x, s):
    if s == 0:
        return x
    return jnp.concatenate([x[s:], x[:s]], axis=0)


def _off_kernel(aux_ref, spa_ref, w01_ref, b01_ref, wd_ref, bd_ref,
                wr_ref, br_ref, wo_ref, bo_ref, o_ref, *, H, W):
    c = spa_ref.shape[-1]
    hw = H * W
    a = jnp.dot(aux_ref[0, 0], w01_ref[0, 0], preferred_element_type=F32)
    a = a + jnp.dot(spa_ref[0], w01_ref[0, 1], preferred_element_type=F32)
    a = a + b01_ref[0]
    offf = jnp.where(a >= 0, a, SLOPE * a)
    r = offf + br_ref[0]
    iot = jax.lax.broadcasted_iota(jnp.int32, (hw, 1), 0)
    hh = iot // W
    ww = iot % W
    for di, d in enumerate((1, 2, 4)):
        hd = jnp.zeros((hw, c), F32) + bd_ref[0][:, di * c:(di + 1) * c]
        for t in range(9):
            dy = (t // 3 - 1) * d
            dx = (t % 3 - 1) * d
            valid = ((hh >= -dy) & (hh <= H - 1 - dy)
                     & (ww >= -dx) & (ww <= W - 1 - dx))
            tap = _rot(offf, dy * W + dx) * valid.astype(F32)
            hd = hd + jnp.dot(tap.astype(BF), wd_ref[0, di * 9 + t],
                              preferred_element_type=F32)
        hd = jnp.where(hd >= 0, hd, SLOPE * hd)
        r = r + jnp.dot(hd.astype(BF), wr_ref[0, di],
                        preferred_element_type=F32)
    o_ref[0, 0] = (jnp.dot(r.astype(BF), wo_ref[0],
                           preferred_element_type=F32) + bo_ref[0])


def _offset_head(aux, spa, w01, b01, wd, bd, wr, br, wo, bo, H, W):
    g, bb, hw, c = aux.shape
    n = wo.shape[-1]
    kfn = functools.partial(_off_kernel, H=H, W=W)
    return pl.pallas_call(
        kfn,
        out_shape=jax.ShapeDtypeStruct((g, bb, hw, n), F32),
        grid=(g, bb),
        in_specs=[
            pl.BlockSpec((1, 1, hw, c), lambda gi, bi: (gi, bi, 0, 0)),
            pl.BlockSpec((1, hw, c), lambda gi, bi: (bi, 0, 0)),
            pl.BlockSpec((1, 2, c, c), lambda gi, bi: (gi, 0, 0, 0)),
            pl.BlockSpec((1, 1, c), lambda gi, bi: (gi, 0, 0)),
            pl.BlockSpec((1, 27, c, c), lambda gi, bi: (gi, 0, 0, 0)),
            pl.BlockSpec((1, 1, 3 * c), lambda gi, bi: (gi, 0, 0)),
            pl.BlockSpec((1, 3, c, c), lambda gi, bi: (gi, 0, 0, 0)),
            pl.BlockSpec((1, 1, c), lambda gi, bi: (gi, 0, 0)),
            pl.BlockSpec((1, c, n), lambda gi, bi: (gi, 0, 0)),
            pl.BlockSpec((1, 1, n), lambda gi, bi: (gi, 0, 0)),
        ],
        out_specs=pl.BlockSpec((1, 1, hw, n), lambda gi, bi: (gi, bi, 0, 0)),
        compiler_params=pltpu.CompilerParams(
            dimension_semantics=("parallel", "parallel")),
    )(aux, spa, w01, b01, wd, bd, wr, br, wo, bo)


def _fuse3_kernel(spa_ref, al_ref, w_ref, b_ref, o_ref):
    acc = jnp.dot(spa_ref[...], w_ref[0], preferred_element_type=F32)
    acc = acc + jnp.dot(al_ref[0], w_ref[1], preferred_element_type=F32)
    acc = acc + jnp.dot(al_ref[1], w_ref[2], preferred_element_type=F32)
    o_ref[...] = acc + b_ref[...]


def _fuse3(spa, al, w, b, tm=512):
    m, c = spa.shape
    tmm = min(m, tm)
    return pl.pallas_call(
        _fuse3_kernel,
        out_shape=jax.ShapeDtypeStruct((m, c), F32),
        grid=(pl.cdiv(m, tmm),),
        in_specs=[
            pl.BlockSpec((tmm, c), lambda i: (i, 0)),
            pl.BlockSpec((2, tmm, c), lambda i: (0, i, 0)),
            pl.BlockSpec((3, c, c), lambda i: (0, 0, 0)),
            pl.BlockSpec((1, c), lambda i: (0, 0)),
        ],
        out_specs=pl.BlockSpec((tmm, c), lambda i: (i, 0)),
        compiler_params=pltpu.CompilerParams(
            dimension_semantics=("parallel",)),
    )(spa, al, w, b)


def _ang_kernel(x_ref, w_ref, b_ref, wf_ref, bf_ref, o_ref):
    acc = jnp.zeros(o_ref.shape, F32) + bf_ref[...]
    for t in range(3):
        a = jnp.dot(x_ref[t], w_ref[t], preferred_element_type=F32) + b_ref[t]
        a = jnp.where(a >= 0, a, SLOPE * a).astype(BF)
        acc = acc + jnp.dot(a, wf_ref[t], preferred_element_type=F32)
    o_ref[...] = acc


def _ang_fuse(x, w, b, wf, bf, tm=512):
    _, m, k = x.shape
    c = w.shape[-1]
    tmm = min(m, tm)
    return pl.pallas_call(
        _ang_kernel,
        out_shape=jax.ShapeDtypeStruct((m, c), F32),
        grid=(pl.cdiv(m, tmm),),
        in_specs=[
            pl.BlockSpec((3, tmm, k), lambda i: (0, i, 0)),
            pl.BlockSpec((3, k, c), lambda i: (0, 0, 0)),
            pl.BlockSpec((3, 1, c), lambda i: (0, 0, 0)),
            pl.BlockSpec((3, c, c), lambda i: (0, 0, 0)),
            pl.BlockSpec((1, c), lambda i: (0, 0)),
        ],
        out_specs=pl.BlockSpec((tmm, c), lambda i: (i, 0)),
        compiler_params=pltpu.CompilerParams(
            dimension_semantics=("parallel",)),
    )(x, w, b, wf, bf)


def _deform_kernel(x_ref, off_ref, w_ref, b_ref, o_ref, *, H, W, TM):
    hw = x_ref.shape[-2]
    c = x_ref.shape[-1]
    xim = x_ref[0, 0]
    off = off_ref[0, 0]
    base = pl.program_id(2) * TM
    rows = base + jax.lax.broadcasted_iota(jnp.int32, (TM, 1), 0)
    hh = (rows // W).astype(F32)
    ww = (rows % W).astype(F32)
    gr = jax.lax.broadcasted_iota(jnp.int32, (1, hw), 1)
    giy = (gr // W).astype(F32)
    gjx = (gr % W).astype(F32)
    acc = jnp.zeros((TM, c), F32) + b_ref[0]
    for t in range(9):
        py = hh - 1.0 + (t // 3) + off[:, 2 * t:2 * t + 1]
        px = ww - 1.0 + (t % 3) + off[:, 2 * t + 1:2 * t + 2]
        ay = jnp.maximum(1.0 - jnp.abs(py - giy), 0.0)
        ax = jnp.maximum(1.0 - jnp.abs(px - gjx), 0.0)
        mask = ay * ax
        tap = jnp.dot(mask.astype(BF), xim, preferred_element_type=F32)
        acc = acc + jnp.dot(tap.astype(BF), w_ref[0, t],
                            preferred_element_type=F32)
    o_ref[0, 0] = jnp.where(acc >= 0, acc, SLOPE * acc).astype(BF)


def _deform_align(x, off, w, b, H, W):
    g, bb, hw, c = x.shape
    tm = min(hw, 256)
    kfn = functools.partial(_deform_kernel, H=H, W=W, TM=tm)
    return pl.pallas_call(
        kfn,
        out_shape=jax.ShapeDtypeStruct((g, bb, hw, c), BF),
        grid=(g, bb, hw // tm),
        in_specs=[
            pl.BlockSpec((1, 1, hw, c), lambda gi, bi, i: (gi, bi, 0, 0)),
            pl.BlockSpec((1, 1, tm, 18), lambda gi, bi, i: (gi, bi, i, 0)),
            pl.BlockSpec((1, 9, c, c), lambda gi, bi, i: (gi, 0, 0, 0)),
            pl.BlockSpec((1, 1, c), lambda gi, bi, i: (gi, 0, 0)),
        ],
        out_specs=pl.BlockSpec((1, 1, tm, c), lambda gi, bi, i: (gi, bi, i, 0)),
        compiler_params=pltpu.CompilerParams(
            dimension_semantics=("parallel", "parallel", "parallel")),
    )(x, off, w, b)


def _stage(P, spa, aux1, aux2):
    b, h, w, c = spa.shape
    m = b * h * w
    bs = b // 4
    m4 = bs * h * w

    aux = jnp.stack([aux1, aux2]).astype(BF).reshape(2, b, h * w, c)
    spar = spa.reshape(b, h * w, c)

    w01 = jnp.stack([_f1x1(P["a1"]["c01"][0]).reshape(2, c, c),
                     _f1x1(P["a2"]["c01"][0]).reshape(2, c, c)])
    b01 = jnp.stack([P["a1"]["c01"][1], P["a2"]["c01"][1]])[:, None, :]
    wd = jnp.stack([jnp.concatenate(
        [_fconv(P[a]["aspp"][i][0]).reshape(9, c, c) for i in range(3)])
        for a in ("a1", "a2")])
    bd = jnp.stack([jnp.concatenate([P[a]["aspp"][i][1] for i in range(3)])
                    for a in ("a1", "a2")])[:, None, :]
    wr = jnp.stack([_f1x1(P[a]["cr"][0]).reshape(3, c, c)
                    for a in ("a1", "a2")])
    br = jnp.stack([P[a]["cr"][1] for a in ("a1", "a2")])[:, None, :]
    wo = jnp.stack([_f1x1(P[a]["c02"][0]) for a in ("a1", "a2")])
    bo = jnp.stack([P[a]["c02"][1] for a in ("a1", "a2")])[:, None, :]
    offsets = _offset_head(aux, spar, w01, b01, wd, bd, wr, br, wo, bo, h, w)

    wdef = jnp.stack([_fconv(P[a]["def"][0]).reshape(9, c, c)
                      for a in ("a1", "a2")])
    bdef = jnp.stack([P[a]["def"][1] for a in ("a1", "a2")])[:, None, :]
    aligned = _deform_align(aux, offsets, wdef, bdef, h, w).reshape(2, m, c)

    wf = _f1x1(P["fuse"][0]).reshape(3, c, c)
    spa_fuse = _fuse3(spa.reshape(m, c), aligned, wf,
                      P["fuse"][1][None, :])

    def amat(t):
        tr = t.reshape(bs, 4, h, w, c).transpose(0, 2, 3, 4, 1)
        return tr.reshape(m4, c * 4)

    al = aligned.reshape(2, b, h, w, c)
    xang = jnp.stack([amat(spa_fuse.astype(BF)),
                      amat(al[0]), amat(al[1])])
    wang = jnp.stack([P["ang"][0].reshape(c, c * 4).T,
                      P["ax1"][0].reshape(c, c * 4).T,
                      P["ax2"][0].reshape(c, c * 4).T]).astype(BF)
    bang = jnp.stack([P["ang"][1], P["ax1"][1], P["ax2"][1]])[:, None, :]
    wf0 = _f1x1(P["fuse0"][0]).reshape(3, c, c)
    ang_fused = _ang_fuse(xang, wang, bang, wf0, P["fuse0"][1][None, :])

    col_gb, _, _ = _patches(ang_fused.astype(BF).reshape(bs, h, w, c),
                            3, 3, 1, 1)
    wgb = jnp.concatenate([_fconv(P["e1"][0]), _fconv(P["e2"][0])], axis=1)
    bgb = jnp.concatenate([P["e1"][1], P["e2"][1]])
    gb = _mm(col_gb[None], wgb[None], bgb[None, None])[0]
    gb = gb.reshape(bs, h, w, 2 * c)
    gamma, beta = gb[..., :c], gb[..., c:]
    out = (spa_fuse.reshape(bs, 4, h, w, c) * gamma[:, None] + beta[:, None])
    return out.reshape(b, h, w, c)


def _conv_s2(x, wgt, bias):
    n = x.shape[0]
    co = wgt.shape[0]
    col, ho, wo = _patches(x, 3, 3, 2, 1)
    out = _mm(col[None], _fconv(wgt)[None], bias[None, None],
              act=True, out_dtype=BF)[0]
    return out.reshape(n, ho, wo, co)


def _stage_params(d, s):
    def cv(p):
        return (d[p + "_w"], d[p + "_b"])

    def al(a):
        p = s + "_" + a
        return {"c01": cv(p + "_conv01"),
                "aspp": [cv(p + "_aspp_c1"), cv(p + "_aspp_c2"),
                         cv(p + "_aspp_c3")],
                "cr": cv(p + "_aspp_cr"),
                "c02": cv(p + "_conv02"),
                "def": cv(p + "_deform")}

    return {"a1": al("align1"), "a2": al("align2"),
            "fuse": cv(s + "_fuse"), "ang": cv(s + "_ang"),
            "ax1": cv(s + "_aux_ang1"), "ax2": cv(s + "_aux_ang2"),
            "fuse0": cv(s + "_fuse0"),
            "e1": cv(s + "_embed_c1"), "e2": cv(s + "_embed_c2")}


def kernel(*args):
    d = dict(zip(_NAMES, args, strict=True))
    x = jnp.transpose(d["in_x"], (0, 2, 3, 1)).astype(BF)
    a1 = [jnp.transpose(d["aux_spa1_%d" % i], (0, 2, 3, 1)) for i in range(3)]
    a2 = [jnp.transpose(d["aux_spa2_%d" % i], (0, 2, 3, 1)) for i in range(3)]

    spa1 = _conv_s2(x, d["spa1_w"], d["spa1_b"])
    f1 = _stage(_stage_params(d, "s1"), spa1, a1[0], a2[0])
    spa2 = _conv_s2(f1.astype(BF), d["spa2_w"], d["spa2_b"])
    f2 = _stage(_stage_params(d, "s2"), spa2, a1[1], a2[1])
    spa3 = _conv_s2(f2.astype(BF), d["spa3_w"], d["spa3_b"])
    f3 = _stage(_stage_params(d, "s3"), spa3, a1[2], a2[2])
    return [jnp.transpose(f, (0, 3, 1, 2)) for f in (f1, f2, f3)]


# speedup vs baseline: 1.0477x; 1.0477x over previous
import functools

import jax
import jax.numpy as jnp
from jax.experimental import pallas as pl
from jax.experimental.pallas import tpu as pltpu

SLOPE = 0.1
BF = jnp.bfloat16
F32 = jnp.float32


def _argnames():
    ns = []
    for s in ("spa1", "spa2", "spa3"):
        ns += [s + "_w", s + "_b"]
    for s in ("s1", "s2", "s3"):
        for al in ("align1", "align2"):
            p = s + "_" + al + "_"
            ns += [p + "conv01_w", p + "conv01_b"]
            for cc in ("c1", "c2", "c3", "cr"):
                ns += [p + "aspp_" + cc + "_w", p + "aspp_" + cc + "_b"]
            ns += [p + "conv02_w", p + "conv02_b",
                   p + "deform_w", p + "deform_b"]
        for t in ("fuse", "aux_ang1", "aux_ang2", "ang", "fuse0",
                  "embed_c1", "embed_c2"):
            ns += [s + "_" + t + "_w", s + "_" + t + "_b"]
    ns += ["in_x", "aux_spa1_0", "aux_spa1_1", "aux_spa1_2",
           "aux_spa2_0", "aux_spa2_1", "aux_spa2_2"]
    return tuple(ns)


_NAMES = _argnames()


def _f1x1(w):
    return w.reshape(w.shape[0], w.shape[1]).T.astype(BF)


def _fconv(w):
    co, ci, kh, kw = w.shape
    return jnp.transpose(w, (2, 3, 1, 0)).reshape(kh * kw * ci, co).astype(BF)


def _patches(x, kh, kw, stride=1, pad=0):
    n, h, w, c = x.shape
    xp = jnp.pad(x, ((0, 0), (pad, pad), (pad, pad), (0, 0)))
    ho = (h + 2 * pad - kh) // stride + 1
    wo = (w + 2 * pad - kw) // stride + 1
    taps = [xp[:, i:i + stride * (ho - 1) + 1:stride,
               j:j + stride * (wo - 1) + 1:stride, :].reshape(n * ho * wo, c)
            for i in range(kh) for j in range(kw)]
    return jnp.concatenate(taps, axis=1), ho, wo


def _mm_kernel(x_ref, w_ref, b_ref, o_ref, *, act):
    acc = jnp.dot(x_ref[0], w_ref[0], preferred_element_type=F32) + b_ref[0]
    if act:
        acc = jnp.where(acc >= 0, acc, SLOPE * acc)
    o_ref[0] = acc.astype(o_ref.dtype)


def _mm(x, w, b, act=False, out_dtype=F32, tm=512):
    g, m, k = x.shape
    n = w.shape[-1]
    tmm = min(m, tm)
    return pl.pallas_call(
        functools.partial(_mm_kernel, act=act),
        out_shape=jax.ShapeDtypeStruct((g, m, n), out_dtype),
        grid=(g, pl.cdiv(m, tmm)),
        in_specs=[
            pl.BlockSpec((1, tmm, k), lambda gi, i: (gi, i, 0)),
            pl.BlockSpec((1, k, n), lambda gi, i: (gi, 0, 0)),
            pl.BlockSpec((1, 1, n), lambda gi, i: (gi, 0, 0)),
        ],
        out_specs=pl.BlockSpec((1, tmm, n), lambda gi, i: (gi, i, 0)),
        compiler_params=pltpu.CompilerParams(
            dimension_semantics=("parallel", "parallel")),
    )(x, w, b)


def _rot(x, s):
    if s == 0:
        return x
    return jnp.concatenate([x[s:], x[:s]], axis=0)


def _off_kernel(aux_ref, spa_ref, w01_ref, b01_ref, wd_ref, bd_ref,
                wr_ref, br_ref, wo_ref, bo_ref, o_ref, *, H, W):
    c = spa_ref.shape[-1]
    hw = H * W
    a = jnp.dot(aux_ref[0, 0], w01_ref[0, 0], preferred_element_type=F32)
    a = a + jnp.dot(spa_ref[0], w01_ref[0, 1], preferred_element_type=F32)
    a = a + b01_ref[0]
    offf = jnp.where(a >= 0, a, SLOPE * a)
    r = offf + br_ref[0]
    iot = jax.lax.broadcasted_iota(jnp.int32, (hw, 1), 0)
    hh = iot // W
    ww = iot % W
    for di, d in enumerate((1, 2, 4)):
        hd = jnp.zeros((hw, c), F32) + bd_ref[0][:, di * c:(di + 1) * c]
        for t in range(9):
            dy = (t // 3 - 1) * d
            dx = (t % 3 - 1) * d
            valid = ((hh >= -dy) & (hh <= H - 1 - dy)
                     & (ww >= -dx) & (ww <= W - 1 - dx))
            tap = _rot(offf, dy * W + dx) * valid.astype(F32)
            hd = hd + jnp.dot(tap.astype(BF), wd_ref[0, di * 9 + t],
                              preferred_element_type=F32)
        hd = jnp.where(hd >= 0, hd, SLOPE * hd)
        r = r + jnp.dot(hd.astype(BF), wr_ref[0, di],
                        preferred_element_type=F32)
    o_ref[0, 0] = (jnp.dot(r.astype(BF), wo_ref[0],
                           preferred_element_type=F32) + bo_ref[0])


def _offset_head(aux, spa, w01, b01, wd, bd, wr, br, wo, bo, H, W):
    g, bb, hw, c = aux.shape
    n = wo.shape[-1]
    kfn = functools.partial(_off_kernel, H=H, W=W)
    return pl.pallas_call(
        kfn,
        out_shape=jax.ShapeDtypeStruct((g, bb, hw, n), F32),
        grid=(g, bb),
        in_specs=[
            pl.BlockSpec((1, 1, hw, c), lambda gi, bi: (gi, bi, 0, 0)),
            pl.BlockSpec((1, hw, c), lambda gi, bi: (bi, 0, 0)),
            pl.BlockSpec((1, 2, c, c), lambda gi, bi: (gi, 0, 0, 0)),
            pl.BlockSpec((1, 1, c), lambda gi, bi: (gi, 0, 0)),
            pl.BlockSpec((1, 27, c, c), lambda gi, bi: (gi, 0, 0, 0)),
            pl.BlockSpec((1, 1, 3 * c), lambda gi, bi: (gi, 0, 0)),
            pl.BlockSpec((1, 3, c, c), lambda gi, bi: (gi, 0, 0, 0)),
            pl.BlockSpec((1, 1, c), lambda gi, bi: (gi, 0, 0)),
            pl.BlockSpec((1, c, n), lambda gi, bi: (gi, 0, 0)),
            pl.BlockSpec((1, 1, n), lambda gi, bi: (gi, 0, 0)),
        ],
        out_specs=pl.BlockSpec((1, 1, hw, n), lambda gi, bi: (gi, bi, 0, 0)),
        compiler_params=pltpu.CompilerParams(
            dimension_semantics=("parallel", "parallel")),
    )(aux, spa, w01, b01, wd, bd, wr, br, wo, bo)


def _fuse3_kernel(spa_ref, al_ref, w_ref, b_ref, o_ref):
    acc = jnp.dot(spa_ref[...], w_ref[0], preferred_element_type=F32)
    acc = acc + jnp.dot(al_ref[0], w_ref[1], preferred_element_type=F32)
    acc = acc + jnp.dot(al_ref[1], w_ref[2], preferred_element_type=F32)
    o_ref[...] = acc + b_ref[...]


def _fuse3(spa, al, w, b, tm=512):
    m, c = spa.shape
    tmm = min(m, tm)
    return pl.pallas_call(
        _fuse3_kernel,
        out_shape=jax.ShapeDtypeStruct((m, c), F32),
        grid=(pl.cdiv(m, tmm),),
        in_specs=[
            pl.BlockSpec((tmm, c), lambda i: (i, 0)),
            pl.BlockSpec((2, tmm, c), lambda i: (0, i, 0)),
            pl.BlockSpec((3, c, c), lambda i: (0, 0, 0)),
            pl.BlockSpec((1, c), lambda i: (0, 0)),
        ],
        out_specs=pl.BlockSpec((tmm, c), lambda i: (i, 0)),
        compiler_params=pltpu.CompilerParams(
            dimension_semantics=("parallel",)),
    )(spa, al, w, b)


def _ang_kernel(xs_ref, al_ref, w_ref, b_ref, wf_ref, bf_ref, o_ref):
    acc = jnp.zeros(o_ref.shape[1:], F32) + bf_ref[...]
    for t in range(3):
        a = jnp.zeros(o_ref.shape[1:], F32) + b_ref[t]
        for v in range(4):
            x = xs_ref[0, v] if t == 0 else al_ref[t - 1, 0, v]
            a = a + jnp.dot(x, w_ref[t, v], preferred_element_type=F32)
        a = jnp.where(a >= 0, a, SLOPE * a).astype(BF)
        acc = acc + jnp.dot(a, wf_ref[t], preferred_element_type=F32)
    o_ref[0] = acc


def _ang_fuse(xs, al, w, b, wf, bf, tm=512):
    bs, _, hw, c = xs.shape
    tmm = min(hw, tm)
    return pl.pallas_call(
        _ang_kernel,
        out_shape=jax.ShapeDtypeStruct((bs, hw, c), F32),
        grid=(bs, pl.cdiv(hw, tmm)),
        in_specs=[
            pl.BlockSpec((1, 4, tmm, c), lambda s, i: (s, 0, i, 0)),
            pl.BlockSpec((2, 1, 4, tmm, c), lambda s, i: (0, s, 0, i, 0)),
            pl.BlockSpec((3, 4, c, c), lambda s, i: (0, 0, 0, 0)),
            pl.BlockSpec((3, 1, c), lambda s, i: (0, 0, 0)),
            pl.BlockSpec((3, c, c), lambda s, i: (0, 0, 0)),
            pl.BlockSpec((1, c), lambda s, i: (0, 0)),
        ],
        out_specs=pl.BlockSpec((1, tmm, c), lambda s, i: (s, i, 0)),
        compiler_params=pltpu.CompilerParams(
            dimension_semantics=("parallel", "parallel")),
    )(xs, al, w, b, wf, bf)


def _deform_kernel(x_ref, off_ref, w_ref, b_ref, o_ref, *, H, W, TM):
    hw = x_ref.shape[-2]
    c = x_ref.shape[-1]
    xim = x_ref[0, 0]
    off = off_ref[0, 0]
    base = pl.program_id(2) * TM
    rows = base + jax.lax.broadcasted_iota(jnp.int32, (TM, 1), 0)
    hh = (rows // W).astype(F32)
    ww = (rows % W).astype(F32)
    gr = jax.lax.broadcasted_iota(jnp.int32, (1, hw), 1)
    giy = (gr // W).astype(F32)
    gjx = (gr % W).astype(F32)
    acc = jnp.zeros((TM, c), F32) + b_ref[0]
    for t in range(9):
        py = hh - 1.0 + (t // 3) + off[:, 2 * t:2 * t + 1]
        px = ww - 1.0 + (t % 3) + off[:, 2 * t + 1:2 * t + 2]
        ay = jnp.maximum(1.0 - jnp.abs(py - giy), 0.0)
        ax = jnp.maximum(1.0 - jnp.abs(px - gjx), 0.0)
        mask = ay * ax
        tap = jnp.dot(mask.astype(BF), xim, preferred_element_type=F32)
        acc = acc + jnp.dot(tap.astype(BF), w_ref[0, t],
                            preferred_element_type=F32)
    o_ref[0, 0] = jnp.where(acc >= 0, acc, SLOPE * acc).astype(BF)


def _deform_align(x, off, w, b, H, W):
    g, bb, hw, c = x.shape
    tm = min(hw, 256)
    kfn = functools.partial(_deform_kernel, H=H, W=W, TM=tm)
    return pl.pallas_call(
        kfn,
        out_shape=jax.ShapeDtypeStruct((g, bb, hw, c), BF),
        grid=(g, bb, hw // tm),
        in_specs=[
            pl.BlockSpec((1, 1, hw, c), lambda gi, bi, i: (gi, bi, 0, 0)),
            pl.BlockSpec((1, 1, tm, 18), lambda gi, bi, i: (gi, bi, i, 0)),
            pl.BlockSpec((1, 9, c, c), lambda gi, bi, i: (gi, 0, 0, 0)),
            pl.BlockSpec((1, 1, c), lambda gi, bi, i: (gi, 0, 0)),
        ],
        out_specs=pl.BlockSpec((1, 1, tm, c), lambda gi, bi, i: (gi, bi, i, 0)),
        compiler_params=pltpu.CompilerParams(
            dimension_semantics=("parallel", "parallel", "parallel")),
    )(x, off, w, b)


def _stage(P, spa, aux1, aux2):
    b, h, w, c = spa.shape
    m = b * h * w
    bs = b // 4
    m4 = bs * h * w

    aux = jnp.stack([aux1, aux2]).astype(BF).reshape(2, b, h * w, c)
    spar = spa.reshape(b, h * w, c)

    w01 = jnp.stack([_f1x1(P["a1"]["c01"][0]).reshape(2, c, c),
                     _f1x1(P["a2"]["c01"][0]).reshape(2, c, c)])
    b01 = jnp.stack([P["a1"]["c01"][1], P["a2"]["c01"][1]])[:, None, :]
    wd = jnp.stack([jnp.concatenate(
        [_fconv(P[a]["aspp"][i][0]).reshape(9, c, c) for i in range(3)])
        for a in ("a1", "a2")])
    bd = jnp.stack([jnp.concatenate([P[a]["aspp"][i][1] for i in range(3)])
                    for a in ("a1", "a2")])[:, None, :]
    wr = jnp.stack([_f1x1(P[a]["cr"][0]).reshape(3, c, c)
                    for a in ("a1", "a2")])
    br = jnp.stack([P[a]["cr"][1] for a in ("a1", "a2")])[:, None, :]
    wo = jnp.stack([_f1x1(P[a]["c02"][0]) for a in ("a1", "a2")])
    bo = jnp.stack([P[a]["c02"][1] for a in ("a1", "a2")])[:, None, :]
    offsets = _offset_head(aux, spar, w01, b01, wd, bd, wr, br, wo, bo, h, w)

    wdef = jnp.stack([_fconv(P[a]["def"][0]).reshape(9, c, c)
                      for a in ("a1", "a2")])
    bdef = jnp.stack([P[a]["def"][1] for a in ("a1", "a2")])[:, None, :]
    aligned = _deform_align(aux, offsets, wdef, bdef, h, w).reshape(2, m, c)

    wf = _f1x1(P["fuse"][0]).reshape(3, c, c)
    spa_fuse = _fuse3(spa.reshape(m, c), aligned, wf,
                      P["fuse"][1][None, :])

    alv = aligned.reshape(2, bs, 4, h * w, c)
    xsv = spa_fuse.astype(BF).reshape(bs, 4, h * w, c)
    wang = jnp.stack([P["ang"][0].reshape(c, c, 4).transpose(2, 1, 0),
                      P["ax1"][0].reshape(c, c, 4).transpose(2, 1, 0),
                      P["ax2"][0].reshape(c, c, 4).transpose(2, 1, 0)]
                     ).astype(BF)
    bang = jnp.stack([P["ang"][1], P["ax1"][1], P["ax2"][1]])[:, None, :]
    wf0 = _f1x1(P["fuse0"][0]).reshape(3, c, c)
    ang_fused = _ang_fuse(xsv, alv, wang, bang, wf0, P["fuse0"][1][None, :])

    col_gb, _, _ = _patches(ang_fused.astype(BF).reshape(bs, h, w, c),
                            3, 3, 1, 1)
    wgb = jnp.concatenate([_fconv(P["e1"][0]), _fconv(P["e2"][0])], axis=1)
    bgb = jnp.concatenate([P["e1"][1], P["e2"][1]])
    gb = _mm(col_gb[None], wgb[None], bgb[None, None])[0]
    gb = gb.reshape(bs, h, w, 2 * c)
    gamma, beta = gb[..., :c], gb[..., c:]
    out = (spa_fuse.reshape(bs, 4, h, w, c) * gamma[:, None] + beta[:, None])
    return out.reshape(b, h, w, c)


def _conv_s2(x, wgt, bias):
    n = x.shape[0]
    co = wgt.shape[0]
    col, ho, wo = _patches(x, 3, 3, 2, 1)
    out = _mm(col[None], _fconv(wgt)[None], bias[None, None],
              act=True, out_dtype=BF)[0]
    return out.reshape(n, ho, wo, co)


def _stage_params(d, s):
    def cv(p):
        return (d[p + "_w"], d[p + "_b"])

    def al(a):
        p = s + "_" + a
        return {"c01": cv(p + "_conv01"),
                "aspp": [cv(p + "_aspp_c1"), cv(p + "_aspp_c2"),
                         cv(p + "_aspp_c3")],
                "cr": cv(p + "_aspp_cr"),
                "c02": cv(p + "_conv02"),
                "def": cv(p + "_deform")}

    return {"a1": al("align1"), "a2": al("align2"),
            "fuse": cv(s + "_fuse"), "ang": cv(s + "_ang"),
            "ax1": cv(s + "_aux_ang1"), "ax2": cv(s + "_aux_ang2"),
            "fuse0": cv(s + "_fuse0"),
            "e1": cv(s + "_embed_c1"), "e2": cv(s + "_embed_c2")}


def kernel(*args):
    d = dict(zip(_NAMES, args, strict=True))
    x = jnp.transpose(d["in_x"], (0, 2, 3, 1)).astype(BF)
    a1 = [jnp.transpose(d["aux_spa1_%d" % i], (0, 2, 3, 1)) for i in range(3)]
    a2 = [jnp.transpose(d["aux_spa2_%d" % i], (0, 2, 3, 1)) for i in range(3)]

    spa1 = _conv_s2(x, d["spa1_w"], d["spa1_b"])
    f1 = _stage(_stage_params(d, "s1"), spa1, a1[0], a2[0])
    spa2 = _conv_s2(f1.astype(BF), d["spa2_w"], d["spa2_b"])
    f2 = _stage(_stage_params(d, "s2"), spa2, a1[1], a2[1])
    spa3 = _conv_s2(f2.astype(BF), d["spa3_w"], d["spa3_b"])
    f3 = _stage(_stage_params(d, "s3"), spa3, a1[2], a2[2])
    return [jnp.transpose(f, (0, 3, 1, 2)) for f in (f1, f2, f3)]
